# Initial kernel scaffold; baseline (speedup 1.0000x reference)
import jax
import jax.numpy as jnp
from jax.experimental import pallas as pl


def kernel(x, Wg, w1, w3, w2):
    raise NotImplementedError("write your pallas kernel here")


# speedup vs baseline: 1.0486x; 1.0486x over previous
import functools

import jax
import jax.numpy as jnp
from jax.experimental import pallas as pl
from jax.experimental.pallas import tpu as pltpu

F_BLK = 512
TOP_K = 2


def _moe_body(x_ref, wg_ref, w1_ref, w3_ref, w2_ref, out_ref, route_ref):
    e = pl.program_id(0)
    fb = pl.program_id(1)

    @pl.when(jnp.logical_and(e == 0, fb == 0))
    def _init():
        xv = x_ref[...]
        logits = jax.lax.dot_general(
            xv, wg_ref[...], (((1,), (0,)), ((), ())),
            preferred_element_type=jnp.float32)
        mx = jnp.max(logits, axis=-1, keepdims=True)
        p = jnp.exp(logits - mx)
        p = p / jnp.sum(p, axis=-1, keepdims=True)
        ecols = jax.lax.broadcasted_iota(jnp.int32, p.shape, 1)
        m1 = jnp.max(p, axis=-1, keepdims=True)
        i1 = jnp.argmax(p, axis=-1)[:, None]
        masked = jnp.where(ecols == i1, -jnp.inf, p)
        m2 = jnp.max(masked, axis=-1, keepdims=True)
        i2 = jnp.argmax(masked, axis=-1)[:, None]
        s = m1 + m2
        route_ref[...] = jnp.where(
            ecols == i1, m1 / s, jnp.where(ecols == i2, m2 / s, 0.0))
        out_ref[...] = jnp.zeros_like(out_ref)

    xv = x_ref[...]
    g = jax.lax.dot_general(
        xv, w1_ref[0], (((1,), (1,)), ((), ())),
        preferred_element_type=jnp.float32)
    u = jax.lax.dot_general(
        xv, w3_ref[0], (((1,), (1,)), ((), ())),
        preferred_element_type=jnp.float32)
    a = (g * jax.lax.logistic(g)) * u
    ecols = jax.lax.broadcasted_iota(jnp.int32, route_ref.shape, 1)
    rw = jnp.sum(jnp.where(ecols == e, route_ref[...], 0.0), axis=1,
                 keepdims=True)
    out_ref[...] += jax.lax.dot_general(
        a * rw, w2_ref[0], (((1,), (1,)), ((), ())),
        preferred_element_type=jnp.float32)


@jax.jit
def kernel(x, Wg, w1, w3, w2):
    m, d = x.shape
    e_num = Wg.shape[1]
    f = w1.shape[1]
    nf = f // F_BLK
    return pl.pallas_call(
        _moe_body,
        grid=(e_num, nf),
        in_specs=[
            pl.BlockSpec((m, d), lambda e, fb: (0, 0)),
            pl.BlockSpec((d, e_num), lambda e, fb: (0, 0)),
            pl.BlockSpec((1, F_BLK, d), lambda e, fb: (e, fb, 0)),
            pl.BlockSpec((1, F_BLK, d), lambda e, fb: (e, fb, 0)),
            pl.BlockSpec((1, d, F_BLK), lambda e, fb: (e, 0, fb)),
        ],
        out_specs=pl.BlockSpec((m, d), lambda e, fb: (0, 0)),
        out_shape=jax.ShapeDtypeStruct((m, d), x.dtype),
        scratch_shapes=[pltpu.VMEM((m, e_num), jnp.float32)],
    )(x, Wg, w1, w3, w2)


# speedup vs baseline: 1.1531x; 1.0996x over previous
import functools

import jax
import jax.numpy as jnp
from jax.experimental import pallas as pl
from jax.experimental.pallas import tpu as pltpu

F_BLK = 1024
TOP_K = 2


def _moe_body(x_ref, wg_ref, w1_ref, w3_ref, w2_ref, out_ref, route_ref):
    e = pl.program_id(0)
    fb = pl.program_id(1)

    @pl.when(jnp.logical_and(e == 0, fb == 0))
    def _init():
        xv = x_ref[...]
        logits = jax.lax.dot_general(
            xv, wg_ref[...], (((1,), (0,)), ((), ())),
            preferred_element_type=jnp.float32)
        mx = jnp.max(logits, axis=-1, keepdims=True)
        p = jnp.exp(logits - mx)
        p = p / jnp.sum(p, axis=-1, keepdims=True)
        ecols = jax.lax.broadcasted_iota(jnp.int32, p.shape, 1)
        m1 = jnp.max(p, axis=-1, keepdims=True)
        i1 = jnp.argmax(p, axis=-1)[:, None]
        masked = jnp.where(ecols == i1, -jnp.inf, p)
        m2 = jnp.max(masked, axis=-1, keepdims=True)
        i2 = jnp.argmax(masked, axis=-1)[:, None]
        s = m1 + m2
        route_ref[...] = jnp.where(
            ecols == i1, m1 / s, jnp.where(ecols == i2, m2 / s, 0.0))
        out_ref[...] = jnp.zeros_like(out_ref)

    xv = x_ref[...]
    g = jax.lax.dot_general(
        xv, w1_ref[0], (((1,), (1,)), ((), ())),
        preferred_element_type=jnp.float32)
    u = jax.lax.dot_general(
        xv, w3_ref[0], (((1,), (1,)), ((), ())),
        preferred_element_type=jnp.float32)
    a = (g * jax.lax.logistic(g)) * u
    ecols = jax.lax.broadcasted_iota(jnp.int32, route_ref.shape, 1)
    rw = jnp.sum(jnp.where(ecols == e, route_ref[...], 0.0), axis=1,
                 keepdims=True)
    out_ref[...] += jax.lax.dot_general(
        a * rw, w2_ref[0], (((1,), (1,)), ((), ())),
        preferred_element_type=jnp.float32)


@jax.jit
def kernel(x, Wg, w1, w3, w2):
    m, d = x.shape
    e_num = Wg.shape[1]
    f = w1.shape[1]
    nf = f // F_BLK
    return pl.pallas_call(
        _moe_body,
        grid=(e_num, nf),
        in_specs=[
            pl.BlockSpec((m, d), lambda e, fb: (0, 0)),
            pl.BlockSpec((d, e_num), lambda e, fb: (0, 0)),
            pl.BlockSpec((1, F_BLK, d), lambda e, fb: (e, fb, 0)),
            pl.BlockSpec((1, F_BLK, d), lambda e, fb: (e, fb, 0)),
            pl.BlockSpec((1, d, F_BLK), lambda e, fb: (e, 0, fb)),
        ],
        out_specs=pl.BlockSpec((m, d), lambda e, fb: (0, 0)),
        out_shape=jax.ShapeDtypeStruct((m, d), x.dtype),
        scratch_shapes=[pltpu.VMEM((m, e_num), jnp.float32)],
    )(x, Wg, w1, w3, w2)
